# baseline (speedup 1.0000x reference)
import jax
import jax.numpy as jnp
from jax.experimental import pallas as pl

K = 32
D = 16
HID = 128
EPS = 1e-12

NODES = 10000
NB = 200
NBK = NB * K
GRID = NODES // NB


def _acos(c):
    ax = jnp.abs(c)
    p = jnp.float32(-0.0012624911)
    p = p * ax + jnp.float32(0.0066700901)
    p = p * ax + jnp.float32(-0.0170881256)
    p = p * ax + jnp.float32(0.0308918810)
    p = p * ax + jnp.float32(-0.0501743046)
    p = p * ax + jnp.float32(0.0889789874)
    p = p * ax + jnp.float32(-0.2145988016)
    p = p * ax + jnp.float32(1.5707963050)
    r = jnp.sqrt(jnp.maximum(1.0 - ax, 0.0)) * p
    return jnp.where(c >= 0, r, jnp.float32(3.14159265358979) - r)


def _fused_kernel(e_ref, w1e_ref, b1_ref, w2_ref, b2_ref, w3_ref, b3_ref,
                  w4_ref, b4_ref, o_ref):
    e = e_ref[...]
    t = e.T
    ts = jnp.roll(t, -1, axis=1)
    sq1 = jnp.sum(t * t, axis=0, keepdims=True) + EPS
    dt = jnp.sum(t * ts, axis=0, keepdims=True)
    sq2 = jnp.roll(sq1, -1, axis=1)
    c = dt * jax.lax.rsqrt(sq1 * sq2)
    c = jnp.clip(c, -1.0, 1.0)
    ang = _acos(c)
    ang3 = ang.T.reshape(NB, K, 1)
    h = jnp.sum(ang3 * w1e_ref[...][None], axis=1) + b1_ref[...]
    h = jnp.tanh(h)
    h = jnp.tanh(jnp.dot(h, w2_ref[...],
                         preferred_element_type=jnp.float32) + b2_ref[...])
    h = jnp.tanh(jnp.dot(h, w3_ref[...],
                         preferred_element_type=jnp.float32) + b3_ref[...])
    o = jax.nn.sigmoid(jnp.dot(h, w4_ref[...],
                               preferred_element_type=jnp.float32) + b4_ref[...])
    o_ref[...] = o


def kernel(x, edge_index, edge_attr, W1, b1, W2, b2, W3, b3, W4, b4):
    del x, edge_index
    W1e = jnp.stack([W1, jnp.zeros_like(W1)], axis=1).reshape(K, HID)
    out = pl.pallas_call(
        _fused_kernel,
        grid=(GRID,),
        in_specs=[
            pl.BlockSpec((NBK, D), lambda i: (i, 0)),
            pl.BlockSpec((K, HID), lambda i: (0, 0)),
            pl.BlockSpec((1, HID), lambda i: (0, 0)),
            pl.BlockSpec((HID, HID), lambda i: (0, 0)),
            pl.BlockSpec((1, HID), lambda i: (0, 0)),
            pl.BlockSpec((HID, HID), lambda i: (0, 0)),
            pl.BlockSpec((1, HID), lambda i: (0, 0)),
            pl.BlockSpec((HID, 1), lambda i: (0, 0)),
            pl.BlockSpec((1, 1), lambda i: (0, 0)),
        ],
        out_specs=pl.BlockSpec((NB, 1), lambda i: (i, 0)),
        out_shape=jax.ShapeDtypeStruct((NODES, 1), jnp.float32),
    )(edge_attr, W1e, b1.reshape(1, HID), W2, b2.reshape(1, HID),
      W3, b3.reshape(1, HID), W4, b4.reshape(1, 1))
    return out[:, 0]


# speedup vs baseline: 1.0437x; 1.0437x over previous
import jax
import jax.numpy as jnp
from jax.experimental import pallas as pl

K = 32
D = 16
HID = 128
EPS = 1e-12

NODES = 10000
NB = 400
NBK = NB * K
GRID = NODES // NB


def _acos(c):
    ax = jnp.abs(c)
    p = jnp.float32(-0.0012624911)
    p = p * ax + jnp.float32(0.0066700901)
    p = p * ax + jnp.float32(-0.0170881256)
    p = p * ax + jnp.float32(0.0308918810)
    p = p * ax + jnp.float32(-0.0501743046)
    p = p * ax + jnp.float32(0.0889789874)
    p = p * ax + jnp.float32(-0.2145988016)
    p = p * ax + jnp.float32(1.5707963050)
    r = jnp.sqrt(jnp.maximum(1.0 - ax, 0.0)) * p
    return jnp.where(c >= 0, r, jnp.float32(3.14159265358979) - r)


def _half_angles(e, w1e, nb_half):
    t = e.T
    ts = jnp.roll(t, -1, axis=1)
    sq1 = jnp.sum(t * t, axis=0, keepdims=True) + EPS
    dt = jnp.sum(t * ts, axis=0, keepdims=True)
    sq2 = jnp.roll(sq1, -1, axis=1)
    c = dt * jax.lax.rsqrt(sq1 * sq2)
    c = jnp.clip(c, -1.0, 1.0)
    ang = _acos(c)
    ang3 = ang.T.reshape(nb_half, K, 1)
    return jnp.sum(ang3 * w1e[None], axis=1)


def _fused_kernel(e1_ref, e2_ref, w1e_ref, b1_ref, w2_ref, b2_ref,
                  w3_ref, b3_ref, w4_ref, b4_ref, o_ref):
    w1e = w1e_ref[...]
    ha = _half_angles(e1_ref[...], w1e, NB // 2)
    hb = _half_angles(e2_ref[...], w1e, NB // 2)
    h = jnp.concatenate([ha, hb], axis=0) + b1_ref[...]
    h = jnp.tanh(h)
    h = jnp.tanh(jnp.dot(h, w2_ref[...],
                         preferred_element_type=jnp.float32) + b2_ref[...])
    h = jnp.tanh(jnp.dot(h, w3_ref[...],
                         preferred_element_type=jnp.float32) + b3_ref[...])
    o = jax.nn.sigmoid(jnp.dot(h, w4_ref[...],
                               preferred_element_type=jnp.float32) + b4_ref[...])
    o_ref[...] = o


def kernel(x, edge_index, edge_attr, W1, b1, W2, b2, W3, b3, W4, b4):
    del x, edge_index
    W1e = jnp.stack([W1, jnp.zeros_like(W1)], axis=1).reshape(K, HID)
    out = pl.pallas_call(
        _fused_kernel,
        grid=(GRID,),
        in_specs=[
            pl.BlockSpec((NBK // 2, D), lambda i: (2 * i, 0)),
            pl.BlockSpec((NBK // 2, D), lambda i: (2 * i + 1, 0)),
            pl.BlockSpec((K, HID), lambda i: (0, 0)),
            pl.BlockSpec((1, HID), lambda i: (0, 0)),
            pl.BlockSpec((HID, HID), lambda i: (0, 0)),
            pl.BlockSpec((1, HID), lambda i: (0, 0)),
            pl.BlockSpec((HID, HID), lambda i: (0, 0)),
            pl.BlockSpec((1, HID), lambda i: (0, 0)),
            pl.BlockSpec((HID, 1), lambda i: (0, 0)),
            pl.BlockSpec((1, 1), lambda i: (0, 0)),
        ],
        out_specs=pl.BlockSpec((NB, 1), lambda i: (i, 0)),
        out_shape=jax.ShapeDtypeStruct((NODES, 1), jnp.float32),
    )(edge_attr, edge_attr, W1e, b1.reshape(1, HID), W2, b2.reshape(1, HID),
      W3, b3.reshape(1, HID), W4, b4.reshape(1, 1))
    return out[:, 0]


# speedup vs baseline: 1.2940x; 1.2398x over previous
import jax
import jax.numpy as jnp
from jax.experimental import pallas as pl

K = 32
D = 16
F = K * D
HID = 128
EPS = 1e-12

NODES = 10000
NN = 1024
GRID = -(-NODES // NN)


def _acos(c):
    ax = jnp.abs(c)
    p = jnp.float32(-0.0012624911)
    p = p * ax + jnp.float32(0.0066700901)
    p = p * ax + jnp.float32(-0.0170881256)
    p = p * ax + jnp.float32(0.0308918810)
    p = p * ax + jnp.float32(-0.0501743046)
    p = p * ax + jnp.float32(0.0889789874)
    p = p * ax + jnp.float32(-0.2145988016)
    p = p * ax + jnp.float32(1.5707963050)
    r = jnp.sqrt(jnp.maximum(1.0 - ax, 0.0)) * p
    return jnp.where(c >= 0, r, jnp.float32(3.14159265358979) - r)


def _fused_kernel(t_ref, sel_ref, w1_ref, b1_ref, w2_ref, b2_ref,
                  w3_ref, b3_ref, w4_ref, b4_ref, o_ref):
    t = t_ref[...]
    tr = jnp.roll(t, -D, axis=1)
    sel = sel_ref[...]
    sq = jnp.dot(t * t, sel,
                 preferred_element_type=jnp.float32) + EPS
    dt = jnp.dot(t * tr, sel,
                 preferred_element_type=jnp.float32)
    sq2 = jnp.roll(sq, -1, axis=1)
    c = dt * jax.lax.rsqrt(sq * sq2)
    c = jnp.clip(c, -1.0, 1.0)
    ang = _acos(c)
    h = jnp.tanh(jnp.dot(ang, w1_ref[...],
                         preferred_element_type=jnp.float32) + b1_ref[...])
    h = jnp.tanh(jnp.dot(h, w2_ref[...],
                         preferred_element_type=jnp.float32) + b2_ref[...])
    h = jnp.tanh(jnp.dot(h, w3_ref[...],
                         preferred_element_type=jnp.float32) + b3_ref[...])
    o = jax.nn.sigmoid(jnp.dot(h, w4_ref[...],
                               preferred_element_type=jnp.float32) + b4_ref[...])
    o_ref[...] = o


def kernel(x, edge_index, edge_attr, W1, b1, W2, b2, W3, b3, W4, b4):
    del x, edge_index
    ea = edge_attr.reshape(NODES, F)
    sel =(jax.lax.broadcasted_iota(jnp.int32, (F, K), 0) // D ==
           jax.lax.broadcasted_iota(jnp.int32, (F, K), 1)).astype(jnp.float32)
    W1e = jnp.zeros((K, HID), jnp.float32).at[0::2].set(W1)
    out = pl.pallas_call(
        _fused_kernel,
        grid=(GRID,),
        in_specs=[
            pl.BlockSpec((NN, F), lambda i: (i, 0)),
            pl.BlockSpec((F, K), lambda i: (0, 0)),
            pl.BlockSpec((K, HID), lambda i: (0, 0)),
            pl.BlockSpec((1, HID), lambda i: (0, 0)),
            pl.BlockSpec((HID, HID), lambda i: (0, 0)),
            pl.BlockSpec((1, HID), lambda i: (0, 0)),
            pl.BlockSpec((HID, HID), lambda i: (0, 0)),
            pl.BlockSpec((1, HID), lambda i: (0, 0)),
            pl.BlockSpec((HID, 1), lambda i: (0, 0)),
            pl.BlockSpec((1, 1), lambda i: (0, 0)),
        ],
        out_specs=pl.BlockSpec((NN, 1), lambda i: (i, 0)),
        out_shape=jax.ShapeDtypeStruct((NODES, 1), jnp.float32),
    )(ea, sel, W1e, b1.reshape(1, HID), W2, b2.reshape(1, HID),
      W3, b3.reshape(1, HID), W4, b4.reshape(1, 1))
    return out[:, 0]
